# baseline (speedup 1.0000x reference)
import functools

import jax
import jax.numpy as jnp
import numpy as np
from jax import lax
from jax.experimental import pallas as pl
from jax.experimental.pallas import tpu as pltpu
from jax.experimental.pallas import tpu_sc as plsc

N_OUT = 64
SEED = 42

NC, NS, L = 2, 16, 16
NW = NC * NS

_ROTS = ((13, 15, 26, 6), (17, 29, 16, 24), (13, 15, 26, 6),
         (17, 29, 16, 24), (13, 15, 26, 6))


def _key_constants(seed):
    def rotl(x, r):
        return np.uint32((int(x) << r | int(x) >> (32 - r)) & 0xFFFFFFFF)

    k1 = np.uint32(seed >> 32)
    k2 = np.uint32(seed & 0xFFFFFFFF)
    ks = [k1, k2, np.uint32(k1 ^ k2 ^ np.uint32(0x1BD11BDA))]
    x0 = np.uint32(ks[0] + np.uint32(0))
    x1 = np.uint32(ks[1] + np.uint32(1))
    for j in range(5):
        for r in _ROTS[j]:
            x0 = np.uint32(x0 + x1)
            x1 = np.uint32(x0 ^ rotl(x1, r))
        x0 = np.uint32(x0 + ks[(j + 1) % 3])
        x1 = np.uint32(x1 + ks[(j + 2) % 3] + np.uint32(j + 1))
    return int(x0), int(x1)


_K1, _K2 = _key_constants(SEED)


def _threefry_bits(k1, k2, lo):
    ks0 = jnp.uint32(k1)
    ks1 = jnp.uint32(k2)
    ks2 = jnp.uint32(k1 ^ k2 ^ 0x1BD11BDA)
    ks = (ks0, ks1, ks2)
    x0 = jnp.zeros((L,), jnp.uint32) + ks0
    x1 = lo + ks1
    for j in range(5):
        for r in _ROTS[j]:
            x0 = x0 + x1
            x1 = x0 ^ ((x1 << jnp.uint32(r)) | (x1 >> jnp.uint32(32 - r)))
        x0 = x0 + ks[(j + 1) % 3]
        x1 = x1 + ks[(j + 2) % 3] + jnp.uint32(j + 1)
    return x0 ^ x1


def _make_onehot(batch):
    rows_per_tile = batch // NW
    vec_steps = rows_per_tile // L

    mesh = plsc.VectorSubcoreMesh(
        core_axis_name="c", subcore_axis_name="s",
        num_cores=NC, num_subcores=NS)

    @functools.partial(
        pl.kernel,
        out_type=jax.ShapeDtypeStruct((N_OUT, batch), jnp.float32),
        mesh=mesh,
        scratch_types=[pltpu.VMEM((N_OUT, rows_per_tile), jnp.float32),
                       pltpu.SemaphoreType.DMA],
        compiler_params=pltpu.CompilerParams(
            needs_layout_passes=False, skip_device_barrier=True,
            use_tc_tiling_on_sc=True),
    )
    def onehot_kernel(out_hbm, buf, sem):
        wid = lax.axis_index("s") * NC + lax.axis_index("c")
        col0 = wid * rows_per_tile

        zeros = jnp.zeros((L,), jnp.float32)
        ones = jnp.ones((L,), jnp.float32)
        lane = lax.iota(jnp.int32, L)
        split = (3 * vec_steps) // 4

        def gen_body(j, _):
            c = j * L
            for e in range(N_OUT):
                buf[e, pl.ds(c, L)] = zeros
            local = c + lane
            lo = (col0 + local).astype(jnp.uint32)
            bits = _threefry_bits(_K1, _K2, lo)
            expert = (bits & jnp.uint32(N_OUT - 1)).astype(jnp.int32)
            plsc.store_scatter(buf, [expert, local], ones)
            return 0

        lax.fori_loop(0, split, gen_body, 0)
        cols_half = split * L
        first = pltpu.async_copy(
            buf.at[:, pl.ds(0, cols_half)],
            out_hbm.at[:, pl.ds(col0, cols_half)], sem)
        lax.fori_loop(split, vec_steps, gen_body, 0)
        pltpu.sync_copy(
            buf.at[:, pl.ds(cols_half, rows_per_tile - cols_half)],
            out_hbm.at[:, pl.ds(col0 + cols_half, rows_per_tile - cols_half)])
        first.wait()

    return onehot_kernel


def kernel(x):
    batch = x.shape[0]
    return _make_onehot(batch)().T


# speedup vs baseline: 1.0052x; 1.0052x over previous
import functools

import jax
import jax.numpy as jnp
import numpy as np
from jax import lax
from jax.experimental import pallas as pl
from jax.experimental.pallas import tpu as pltpu
from jax.experimental.pallas import tpu_sc as plsc

N_OUT = 64
SEED = 42

NC, NS, L = 2, 16, 16
NW = NC * NS

_ROTS = ((13, 15, 26, 6), (17, 29, 16, 24), (13, 15, 26, 6),
         (17, 29, 16, 24), (13, 15, 26, 6))


def _key_constants(seed):
    def rotl(x, r):
        return np.uint32((int(x) << r | int(x) >> (32 - r)) & 0xFFFFFFFF)

    k1 = np.uint32(seed >> 32)
    k2 = np.uint32(seed & 0xFFFFFFFF)
    ks = [k1, k2, np.uint32(k1 ^ k2 ^ np.uint32(0x1BD11BDA))]
    x0 = np.uint32(ks[0] + np.uint32(0))
    x1 = np.uint32(ks[1] + np.uint32(1))
    for j in range(5):
        for r in _ROTS[j]:
            x0 = np.uint32(x0 + x1)
            x1 = np.uint32(x0 ^ rotl(x1, r))
        x0 = np.uint32(x0 + ks[(j + 1) % 3])
        x1 = np.uint32(x1 + ks[(j + 2) % 3] + np.uint32(j + 1))
    return int(x0), int(x1)


_K1, _K2 = _key_constants(SEED)


def _threefry_bits(k1, k2, lo):
    ks0 = jnp.uint32(k1)
    ks1 = jnp.uint32(k2)
    ks2 = jnp.uint32(k1 ^ k2 ^ 0x1BD11BDA)
    ks = (ks0, ks1, ks2)
    x0 = jnp.zeros((L,), jnp.uint32) + ks0
    x1 = lo + ks1
    for j in range(5):
        for r in _ROTS[j]:
            x0 = x0 + x1
            x1 = x0 ^ ((x1 << jnp.uint32(r)) | (x1 >> jnp.uint32(32 - r)))
        x0 = x0 + ks[(j + 1) % 3]
        x1 = x1 + ks[(j + 2) % 3] + jnp.uint32(j + 1)
    return x0 ^ x1


def _make_onehot(batch):
    rows_per_tile = batch // NW
    vec_steps = rows_per_tile // L

    mesh = plsc.VectorSubcoreMesh(
        core_axis_name="c", subcore_axis_name="s",
        num_cores=NC, num_subcores=NS)

    @functools.partial(
        pl.kernel,
        out_type=jax.ShapeDtypeStruct((N_OUT, batch), jnp.float32),
        mesh=mesh,
        scratch_types=[pltpu.VMEM((N_OUT, rows_per_tile), jnp.float32),
                       pltpu.SemaphoreType.DMA],
        compiler_params=pltpu.CompilerParams(
            needs_layout_passes=False, skip_device_barrier=True,
            use_tc_tiling_on_sc=True),
    )
    def onehot_kernel(out_hbm, buf, sem):
        wid = lax.axis_index("s") * NC + lax.axis_index("c")
        col0 = wid * rows_per_tile

        zeros = jnp.zeros((L,), jnp.float32)
        ones = jnp.ones((L,), jnp.float32)
        lane = lax.iota(jnp.int32, L)
        split = vec_steps // 4

        def gen_body(j, _):
            c = j * L
            for e in range(N_OUT):
                buf[e, pl.ds(c, L)] = zeros
            local = c + lane
            lo = (col0 + local).astype(jnp.uint32)
            bits = _threefry_bits(_K1, _K2, lo)
            expert = (bits & jnp.uint32(N_OUT - 1)).astype(jnp.int32)
            plsc.store_scatter(buf, [expert, local], ones)
            return 0

        lax.fori_loop(0, split, gen_body, 0)
        cols_half = split * L
        first = pltpu.async_copy(
            buf.at[:, pl.ds(0, cols_half)],
            out_hbm.at[:, pl.ds(col0, cols_half)], sem)
        lax.fori_loop(split, vec_steps, gen_body, 0)
        pltpu.sync_copy(
            buf.at[:, pl.ds(cols_half, rows_per_tile - cols_half)],
            out_hbm.at[:, pl.ds(col0 + cols_half, rows_per_tile - cols_half)])
        first.wait()

    return onehot_kernel


def kernel(x):
    batch = x.shape[0]
    return _make_onehot(batch)().T


# speedup vs baseline: 1.0127x; 1.0075x over previous
import functools

import jax
import jax.numpy as jnp
import numpy as np
from jax import lax
from jax.experimental import pallas as pl
from jax.experimental.pallas import tpu as pltpu
from jax.experimental.pallas import tpu_sc as plsc

N_OUT = 64
SEED = 42

NC, NS, L = 2, 16, 16
NW = NC * NS

_ROTS = ((13, 15, 26, 6), (17, 29, 16, 24), (13, 15, 26, 6),
         (17, 29, 16, 24), (13, 15, 26, 6))


def _key_constants(seed):
    M = 0xFFFFFFFF

    def rotl(x, r):
        return ((x << r) | (x >> (32 - r))) & M

    k1 = (seed >> 32) & M
    k2 = seed & M
    ks = [k1, k2, k1 ^ k2 ^ 0x1BD11BDA]
    x0 = ks[0] & M
    x1 = (ks[1] + 1) & M
    for j in range(5):
        for r in _ROTS[j]:
            x0 = (x0 + x1) & M
            x1 = x0 ^ rotl(x1, r)
        x0 = (x0 + ks[(j + 1) % 3]) & M
        x1 = (x1 + ks[(j + 2) % 3] + j + 1) & M
    return x0, x1


_K1, _K2 = _key_constants(SEED)


def _threefry_bits(k1, k2, lo):
    ks0 = jnp.uint32(k1)
    ks1 = jnp.uint32(k2)
    ks2 = jnp.uint32(k1 ^ k2 ^ 0x1BD11BDA)
    ks = (ks0, ks1, ks2)
    x0 = jnp.zeros((L,), jnp.uint32) + ks0
    x1 = lo + ks1
    for j in range(5):
        for r in _ROTS[j]:
            x0 = x0 + x1
            x1 = x0 ^ ((x1 << jnp.uint32(r)) | (x1 >> jnp.uint32(32 - r)))
        x0 = x0 + ks[(j + 1) % 3]
        x1 = x1 + ks[(j + 2) % 3] + jnp.uint32(j + 1)
    return x0 ^ x1


def _make_onehot(batch):
    rows_per_tile = batch // NW
    vec_steps = rows_per_tile // L

    mesh = plsc.VectorSubcoreMesh(
        core_axis_name="c", subcore_axis_name="s",
        num_cores=NC, num_subcores=NS)

    @functools.partial(
        pl.kernel,
        out_type=jax.ShapeDtypeStruct((N_OUT, batch), jnp.float32),
        mesh=mesh,
        scratch_types=[pltpu.VMEM((N_OUT, rows_per_tile), jnp.float32),
                       pltpu.SemaphoreType.DMA],
        compiler_params=pltpu.CompilerParams(
            needs_layout_passes=False, skip_device_barrier=True,
            use_tc_tiling_on_sc=True),
    )
    def onehot_kernel(out_hbm, buf, sem):
        wid = lax.axis_index("s") * NC + lax.axis_index("c")
        col0 = wid * rows_per_tile

        zeros = jnp.zeros((L,), jnp.float32)
        ones = jnp.ones((L,), jnp.float32)
        lane = lax.iota(jnp.int32, L)
        half = vec_steps // 2

        def gen_body(j, _):
            c = j * L
            for e in range(N_OUT):
                buf[e, pl.ds(c, L)] = zeros
            local = c + lane
            lo = (col0 + local).astype(jnp.uint32)
            bits = _threefry_bits(_K1, _K2, lo)
            expert = (bits & jnp.uint32(N_OUT - 1)).astype(jnp.int32)
            plsc.store_scatter(buf, [expert, local], ones)
            return 0

        lax.fori_loop(0, half, gen_body, 0)
        cols_half = half * L
        first = pltpu.async_copy(
            buf.at[:, pl.ds(0, cols_half)],
            out_hbm.at[:, pl.ds(col0, cols_half)], sem)
        lax.fori_loop(half, vec_steps, gen_body, 0)
        pltpu.sync_copy(
            buf.at[:, pl.ds(cols_half, rows_per_tile - cols_half)],
            out_hbm.at[:, pl.ds(col0 + cols_half, rows_per_tile - cols_half)])
        first.wait()

    return onehot_kernel


def kernel(x):
    batch = x.shape[0]
    return _make_onehot(batch)().T


# speedup vs baseline: 1.0154x; 1.0026x over previous
import functools

import jax
import jax.numpy as jnp
import numpy as np
from jax import lax
from jax.experimental import pallas as pl
from jax.experimental.pallas import tpu as pltpu
from jax.experimental.pallas import tpu_sc as plsc

N_OUT = 64
SEED = 42

NC, NS, L = 2, 16, 16
NW = NC * NS

_ROTS = ((13, 15, 26, 6), (17, 29, 16, 24), (13, 15, 26, 6),
         (17, 29, 16, 24), (13, 15, 26, 6))


def _key_constants(seed):
    M = 0xFFFFFFFF

    def rotl(x, r):
        return ((x << r) | (x >> (32 - r))) & M

    k1 = (seed >> 32) & M
    k2 = seed & M
    ks = [k1, k2, k1 ^ k2 ^ 0x1BD11BDA]
    x0 = ks[0] & M
    x1 = (ks[1] + 1) & M
    for j in range(5):
        for r in _ROTS[j]:
            x0 = (x0 + x1) & M
            x1 = x0 ^ rotl(x1, r)
        x0 = (x0 + ks[(j + 1) % 3]) & M
        x1 = (x1 + ks[(j + 2) % 3] + j + 1) & M
    return x0, x1


_K1, _K2 = _key_constants(SEED)


def _threefry_bits(k1, k2, lo):
    ks0 = jnp.uint32(k1)
    ks1 = jnp.uint32(k2)
    ks2 = jnp.uint32(k1 ^ k2 ^ 0x1BD11BDA)
    ks = (ks0, ks1, ks2)
    x0 = jnp.zeros((L,), jnp.uint32) + ks0
    x1 = lo + ks1
    for j in range(5):
        for r in _ROTS[j]:
            x0 = x0 + x1
            x1 = x0 ^ ((x1 << jnp.uint32(r)) | (x1 >> jnp.uint32(32 - r)))
        x0 = x0 + ks[(j + 1) % 3]
        x1 = x1 + ks[(j + 2) % 3] + jnp.uint32(j + 1)
    return x0 ^ x1


def _make_onehot(batch):
    rows_per_tile = batch // NW
    vec_steps = rows_per_tile // L

    mesh = plsc.VectorSubcoreMesh(
        core_axis_name="c", subcore_axis_name="s",
        num_cores=NC, num_subcores=NS)

    @functools.partial(
        pl.kernel,
        out_type=jax.ShapeDtypeStruct((N_OUT, batch), jnp.float32),
        mesh=mesh,
        scratch_types=[pltpu.VMEM((N_OUT, rows_per_tile), jnp.float32),
                       pltpu.SemaphoreType.DMA],
        compiler_params=pltpu.CompilerParams(
            needs_layout_passes=False, skip_device_barrier=True,
            use_tc_tiling_on_sc=True),
    )
    def onehot_kernel(out_hbm, buf, sem):
        wid = lax.axis_index("s") * NC + lax.axis_index("c")
        col0 = wid * rows_per_tile

        zeros = jnp.zeros((L,), jnp.float32)
        ones = jnp.ones((L,), jnp.float32)
        lane = lax.iota(jnp.int32, L)
        half = vec_steps // 2

        def gen_body(j, _):
            c = j * L

            def zrow(i, _):
                for u in range(8):
                    buf[i * 8 + u, pl.ds(c, L)] = zeros
                return 0

            lax.fori_loop(0, N_OUT // 8, zrow, 0)
            local = c + lane
            lo = (col0 + local).astype(jnp.uint32)
            bits = _threefry_bits(_K1, _K2, lo)
            expert = (bits & jnp.uint32(N_OUT - 1)).astype(jnp.int32)
            plsc.store_scatter(buf, [expert, local], ones)
            return 0

        lax.fori_loop(0, half, gen_body, 0)
        cols_half = half * L
        first = pltpu.async_copy(
            buf.at[:, pl.ds(0, cols_half)],
            out_hbm.at[:, pl.ds(col0, cols_half)], sem)
        lax.fori_loop(half, vec_steps, gen_body, 0)
        pltpu.sync_copy(
            buf.at[:, pl.ds(cols_half, rows_per_tile - cols_half)],
            out_hbm.at[:, pl.ds(col0 + cols_half, rows_per_tile - cols_half)])
        first.wait()

    return onehot_kernel


def kernel(x):
    batch = x.shape[0]
    return _make_onehot(batch)().T
